# baseline (speedup 1.0000x reference)
import functools

import jax
import jax.numpy as jnp
from jax import lax
from jax.experimental import pallas as pl
from jax.experimental.pallas import tpu as pltpu
from jax.experimental.pallas import tpu_sc as plsc

_NC = 2
_NS = 16
_NW = _NC * _NS

_CHUNK = 6400
_NBUF = 4
_HBM_EVERY = 3


def _make_sc_gather(n_tab, n_idx):
    per_tile = n_idx // _NW
    n_chunks = per_tile // _CHUNK
    mesh = plsc.VectorSubcoreMesh(core_axis_name="c", subcore_axis_name="s")

    @functools.partial(
        pl.kernel,
        mesh=mesh,
        out_type=jax.ShapeDtypeStruct((n_idx,), jnp.float32),
        scratch_types=[pltpu.VMEM_SHARED((n_tab,), jnp.float32)]
        + [pltpu.VMEM((_CHUNK,), jnp.int32)] * _NBUF
        + [pltpu.VMEM((_CHUNK,), jnp.float32)] * _NBUF
        + [pltpu.SemaphoreType.DMA] * (3 * _NBUF),
    )
    def gather_kernel(table_hbm, idx_hbm, out_hbm, spm, *rest):
        idx_v = rest[0:_NBUF]
        rows_v = rest[_NBUF : 2 * _NBUF]
        sems = rest[2 * _NBUF :]
        sem_i = sems[0:_NBUF]
        sem_g = sems[_NBUF : 2 * _NBUF]
        sem_o = sems[2 * _NBUF :]
        s = lax.axis_index("s")
        wid = s * _NC + lax.axis_index("c")

        @pl.when(s == 0)
        def _stage():
            pltpu.sync_copy(table_hbm, spm)

        plsc.subcore_barrier()
        base = wid * per_tile

        def idx_start(i):
            b = i % _NBUF
            src = idx_hbm.at[pl.ds(base + i * _CHUNK, _CHUNK)]
            return pltpu.async_copy(src, idx_v[b], sem_i[b])

        def gather_start(i):
            b = i % _NBUF
            src = table_hbm if i % _HBM_EVERY == _HBM_EVERY - 1 else spm
            return pltpu.async_copy(src.at[idx_v[b]], rows_v[b], sem_g[b])

        def out_start(i):
            b = i % _NBUF
            dst = out_hbm.at[pl.ds(base + i * _CHUNK, _CHUNK)]
            return pltpu.async_copy(rows_v[b], dst, sem_o[b])

        def elu_rows(b):
            def body(j, _):
                v = rows_v[b][pl.ds(j * 16, 16)]
                rows_v[b][pl.ds(j * 16, 16)] = jnp.where(v > 0, v + 1.0, jnp.exp(v))
                return ()

            lax.fori_loop(0, _CHUNK // 16, body, ())

        cp = {0: idx_start(0)}
        g = {}
        o = {}
        for i in range(n_chunks):
            cp[i].wait()
            if i >= _NBUF:
                o[i - _NBUF].wait()
            g[i] = gather_start(i)
            if i >= 1:
                g[i - 1].wait()
                if i + 1 < n_chunks:
                    cp[i + 1] = idx_start(i + 1)
                elu_rows((i - 1) % _NBUF)
                o[i - 1] = out_start(i - 1)
            elif i + 1 < n_chunks:
                cp[i + 1] = idx_start(i + 1)
        g[n_chunks - 1].wait()
        elu_rows((n_chunks - 1) % _NBUF)
        o[n_chunks - 1] = out_start(n_chunks - 1)
        for j in range(max(0, n_chunks - _NBUF), n_chunks):
            o[j].wait()

    return gather_kernel


def kernel(points, uncertainty):
    b, t = points.shape
    table = uncertainty.reshape(-1)
    idx = points.reshape(-1)
    out = _make_sc_gather(table.shape[0], idx.shape[0])(table, idx)
    return out.reshape(b, t, 1)


# speedup vs baseline: 1.1057x; 1.1057x over previous
import functools

import jax
import jax.numpy as jnp
from jax import lax
from jax.experimental import pallas as pl
from jax.experimental.pallas import tpu as pltpu
from jax.experimental.pallas import tpu_sc as plsc

_NC = 2
_NS = 16
_NW = _NC * _NS

_CHUNK = 3200
_NBUF = 8
_HBM_EVERY = 10**9


def _make_sc_gather(n_tab, n_idx):
    per_tile = n_idx // _NW
    n_chunks = per_tile // _CHUNK
    mesh = plsc.VectorSubcoreMesh(core_axis_name="c", subcore_axis_name="s")

    @functools.partial(
        pl.kernel,
        mesh=mesh,
        out_type=jax.ShapeDtypeStruct((n_idx,), jnp.float32),
        scratch_types=[pltpu.VMEM_SHARED((n_tab,), jnp.float32)]
        + [pltpu.VMEM((_CHUNK,), jnp.int32)] * _NBUF
        + [pltpu.VMEM((_CHUNK,), jnp.float32)] * _NBUF
        + [pltpu.SemaphoreType.DMA] * (3 * _NBUF),
    )
    def gather_kernel(table_hbm, idx_hbm, out_hbm, spm, *rest):
        idx_v = rest[0:_NBUF]
        rows_v = rest[_NBUF : 2 * _NBUF]
        sems = rest[2 * _NBUF :]
        sem_i = sems[0:_NBUF]
        sem_g = sems[_NBUF : 2 * _NBUF]
        sem_o = sems[2 * _NBUF :]
        s = lax.axis_index("s")
        wid = s * _NC + lax.axis_index("c")

        @pl.when(s == 0)
        def _stage():
            pltpu.sync_copy(table_hbm, spm)

        plsc.subcore_barrier()
        base = wid * per_tile

        def idx_start(i):
            b = i % _NBUF
            src = idx_hbm.at[pl.ds(base + i * _CHUNK, _CHUNK)]
            return pltpu.async_copy(src, idx_v[b], sem_i[b])

        def gather_start(i):
            b = i % _NBUF
            src = table_hbm if i % _HBM_EVERY == _HBM_EVERY - 1 else spm
            return pltpu.async_copy(src.at[idx_v[b]], rows_v[b], sem_g[b])

        def out_start(i):
            b = i % _NBUF
            dst = out_hbm.at[pl.ds(base + i * _CHUNK, _CHUNK)]
            return pltpu.async_copy(rows_v[b], dst, sem_o[b])

        def elu_rows(b):
            def body(j, _):
                v = rows_v[b][pl.ds(j * 16, 16)]
                rows_v[b][pl.ds(j * 16, 16)] = jnp.where(v > 0, v + 1.0, jnp.exp(v))
                return ()

            lax.fori_loop(0, _CHUNK // 16, body, ())

        cp = {0: idx_start(0)}
        g = {}
        o = {}
        for i in range(n_chunks):
            cp[i].wait()
            if i >= _NBUF:
                o[i - _NBUF].wait()
            g[i] = gather_start(i)
            if i >= 1:
                g[i - 1].wait()
                if i + 1 < n_chunks:
                    cp[i + 1] = idx_start(i + 1)
                elu_rows((i - 1) % _NBUF)
                o[i - 1] = out_start(i - 1)
            elif i + 1 < n_chunks:
                cp[i + 1] = idx_start(i + 1)
        g[n_chunks - 1].wait()
        elu_rows((n_chunks - 1) % _NBUF)
        o[n_chunks - 1] = out_start(n_chunks - 1)
        for j in range(max(0, n_chunks - _NBUF), n_chunks):
            o[j].wait()

    return gather_kernel


def kernel(points, uncertainty):
    b, t = points.shape
    table = uncertainty.reshape(-1)
    idx = points.reshape(-1)
    out = _make_sc_gather(table.shape[0], idx.shape[0])(table, idx)
    return out.reshape(b, t, 1)
